# baseline (speedup 1.0000x reference)
import functools

import jax
import jax.numpy as jnp
from jax import lax
from jax.experimental import pallas as pl
from jax.experimental.pallas import tpu as pltpu

BV = 64
BO = 512


def _pool_body(nv_ref, x_hbm, o_ref, buf, acc, sems, *, bv, max_views):
    b = pl.program_id(0)
    nv = jnp.minimum(nv_ref[b], max_views)
    nchunks = (nv + bv - 1) // bv

    last_start = jnp.maximum(0, ((nv - bv + 7) // 8) * 8)

    def chunk_row0(i):
        return pl.multiple_of(jnp.minimum(i * bv, last_start), 8)

    def start(i, slot):
        pltpu.make_async_copy(
            x_hbm.at[b, pl.ds(chunk_row0(i), bv), :], buf.at[slot], sems.at[slot]
        ).start()

    acc[...] = jnp.full_like(acc, -jnp.inf)
    start(0, 0)

    def step(i, _):
        slot = lax.rem(i, 2)

        @pl.when(i + 1 < nchunks)
        def _prefetch():
            start(i + 1, 1 - slot)

        pltpu.make_async_copy(
            x_hbm.at[b, pl.ds(0, bv), :], buf.at[slot], sems.at[slot]
        ).wait()
        data = buf[slot]
        row0 = chunk_row0(i)

        @pl.when(row0 + bv <= nv)
        def _full():
            part = data[0:8]
            for r in range(1, bv // 8):
                part = jnp.maximum(part, data[r * 8:(r + 1) * 8])
            acc[...] = jnp.maximum(acc[...], part)

        @pl.when(row0 + bv > nv)
        def _masked():
            row = row0 + lax.broadcasted_iota(jnp.int32, (bv, 1), 0)
            m = jnp.where(row < nv, data, -jnp.inf)
            part = m[0:8]
            for r in range(1, bv // 8):
                part = jnp.maximum(part, m[r * 8:(r + 1) * 8])
            acc[...] = jnp.maximum(acc[...], part)

        return 0

    lax.fori_loop(0, nchunks, step, 0)
    o_ref[0] = jnp.max(acc[...], axis=0, keepdims=True)


def _linear_body(k_ref, w_ref, bias_ref, o_ref):
    out = lax.dot_general(
        k_ref[...], w_ref[...],
        dimension_numbers=(((1,), (1,)), ((), ())),
        preferred_element_type=jnp.float32,
    )
    o_ref[...] = out + bias_ref[...]


def kernel(batch_size, max_num_views, num_views, x, W, b):
    B, V, D = x.shape
    O = W.shape[0]

    pool = pl.pallas_call(
        functools.partial(_pool_body, bv=BV, max_views=V),
        grid_spec=pltpu.PrefetchScalarGridSpec(
            num_scalar_prefetch=1,
            grid=(B,),
            in_specs=[pl.BlockSpec(memory_space=pl.ANY)],
            out_specs=pl.BlockSpec((1, 1, D), lambda bi, nv_ref: (bi, 0, 0)),
            scratch_shapes=[
                pltpu.VMEM((2, BV, D), jnp.float32),
                pltpu.VMEM((8, D), jnp.float32),
                pltpu.SemaphoreType.DMA((2,)),
            ],
        ),
        out_shape=jax.ShapeDtypeStruct((B, 1, D), jnp.float32),
        compiler_params=pltpu.CompilerParams(
            dimension_semantics=("arbitrary",),
        ),
    )
    k = pool(num_views.astype(jnp.int32), x).reshape(B, D)

    bias = b.reshape(1, O)
    linear = pl.pallas_call(
        _linear_body,
        grid=(O // BO,),
        in_specs=[
            pl.BlockSpec((B, D), lambda o: (0, 0)),
            pl.BlockSpec((BO, D), lambda o: (o, 0)),
            pl.BlockSpec((1, BO), lambda o: (0, o)),
        ],
        out_specs=pl.BlockSpec((B, BO), lambda o: (0, o)),
        out_shape=jax.ShapeDtypeStruct((B, O), jnp.float32),
        compiler_params=pltpu.CompilerParams(
            dimension_semantics=("arbitrary",),
        ),
    )
    logits = linear(k, W, bias)
    return (logits, k)
